# Initial kernel scaffold; baseline (speedup 1.0000x reference)
import jax
import jax.numpy as jnp
from jax.experimental import pallas as pl


def kernel(x, edge_index, rel):
    raise NotImplementedError("write your pallas kernel here")


# speedup vs baseline: 6.4623x; 6.4623x over previous
import functools

import jax
import jax.numpy as jnp
from jax import lax
from jax.experimental import pallas as pl
from jax.experimental.pallas import tpu as pltpu
from jax.experimental.pallas import tpu_sc as plsc

_N_NODES = 10000
_N_EDGES = 320000
_DIM = 128
_DIM_R = 64
_PI = 3.141592653589793

_NC = 2
_NS = 16
_NW = _NC * _NS
_L = 16

_EPW = _N_EDGES // _NW
_CHUNK = 80
_NCHUNK = _EPW // _CHUNK


def _fast_sqrt(s):
  i = lax.bitcast_convert_type(s, jnp.int32)
  i = jnp.int32(0x5F3759DF) - lax.shift_right_logical(i, 1)
  y = lax.bitcast_convert_type(i, jnp.float32)
  h = s * 0.5
  y = y * (1.5 - h * y * y)
  y = y * (1.5 - h * y * y)
  return s * y


def _sc_body(x_hbm, u_hbm, v_hbm, rel_hbm, out_hbm,
             u_idx, v_idx, ru, rv, relv, ov, sem):
  wid = lax.axis_index("s") * _NC + lax.axis_index("c")
  base = wid * _EPW
  pltpu.sync_copy(rel_hbm, relv)
  pltpu.sync_copy(u_hbm.at[pl.ds(base, _EPW)], u_idx)
  pltpu.sync_copy(v_hbm.at[pl.ds(base, _EPW)], v_idx)

  rc = [relv[j] for j in range(4)]
  rs = [relv[4 + j] for j in range(4)]
  lane = lax.iota(jnp.int32, _L)
  perms = [lane ^ k for k in (8, 4, 2, 1)]

  def chunk(ci, _):
    off = ci * _CHUNK
    g1 = pltpu.async_copy(x_hbm.at[u_idx.at[pl.ds(off, _CHUNK)]], ru, sem)
    g2 = pltpu.async_copy(x_hbm.at[v_idx.at[pl.ds(off, _CHUNK)]], rv, sem)
    g1.wait()
    g2.wait()

    def group(g, _):
      def edge(k, svec):
        e = g * _L + k
        acc = None
        for j in range(4):
          reu = ru[e, pl.ds(_L * j, _L)]
          imu = ru[e, pl.ds(_DIM_R + _L * j, _L)]
          rev = rv[e, pl.ds(_L * j, _L)]
          imv = rv[e, pl.ds(_DIM_R + _L * j, _L)]
          res = reu * rc[j] - imu * rs[j] - rev
          ims = imu * rc[j] + reu * rs[j] - imv
          t = _fast_sqrt(res * res + ims * ims)
          acc = t if acc is None else acc + t
        for p in perms:
          acc = acc + acc.at[p].get(mode="promise_in_bounds")
        return jnp.where(lane == k, acc, svec)

      svec = lax.fori_loop(0, _L, edge, jnp.zeros((_L,), jnp.float32))
      ov[pl.ds(off + g * _L, _L)] = svec
      return 0

    lax.fori_loop(0, _CHUNK // _L, group, 0)
    return 0

  lax.fori_loop(0, _NCHUNK, chunk, 0)
  pltpu.sync_copy(ov, out_hbm.at[pl.ds(base, _EPW)])


@jax.jit
def kernel(x, edge_index, rel):
  u = edge_index[0].astype(jnp.int32)
  v = edge_index[1].astype(jnp.int32)
  r = rel.reshape(-1).astype(jnp.float32) / _PI
  relbuf = jnp.concatenate([jnp.cos(r), jnp.sin(r)]).reshape(8, _L)

  mesh = plsc.VectorSubcoreMesh(core_axis_name="c", subcore_axis_name="s")
  f = pl.kernel(
      _sc_body,
      out_type=jax.ShapeDtypeStruct((_N_EDGES,), jnp.float32),
      mesh=mesh,
      scratch_types=[
          pltpu.VMEM((_EPW,), jnp.int32),
          pltpu.VMEM((_EPW,), jnp.int32),
          pltpu.VMEM((_CHUNK, _DIM), jnp.float32),
          pltpu.VMEM((_CHUNK, _DIM), jnp.float32),
          pltpu.VMEM((8, _L), jnp.float32),
          pltpu.VMEM((_EPW,), jnp.float32),
          pltpu.SemaphoreType.DMA,
      ],
  )
  return f(x, u, v, relbuf)


# speedup vs baseline: 7.7052x; 1.1923x over previous
import jax
import jax.numpy as jnp
from jax import lax
from jax.experimental import pallas as pl
from jax.experimental.pallas import tpu as pltpu
from jax.experimental.pallas import tpu_sc as plsc

_N_NODES = 10000
_N_EDGES = 320000
_DIM = 128
_DIM_R = 64
_PI = 3.141592653589793

_NC = 2
_NS = 16
_NW = _NC * _NS
_L = 16

_EPW = _N_EDGES // _NW
_CHUNK = 80
_NCHUNK = _EPW // _CHUNK


def _fast_sqrt(s):
  i = lax.bitcast_convert_type(s, jnp.int32)
  i = jnp.int32(0x5F3759DF) - lax.shift_right_logical(i, 1)
  y = lax.bitcast_convert_type(i, jnp.float32)
  h = s * 0.5
  y = y * (1.5 - h * y * y)
  y = y * (1.5 - h * y * y)
  return s * y


def _sc_body(x_hbm, u_hbm, v_hbm, rel_hbm, out_hbm,
             xs, u_idx, v_idx, ru, rv, relv, ov, sem):
  wid = lax.axis_index("s") * _NC + lax.axis_index("c")
  base = wid * _EPW
  pltpu.sync_copy(rel_hbm, relv)
  pltpu.sync_copy(u_hbm.at[pl.ds(base, _EPW)], u_idx)
  pltpu.sync_copy(v_hbm.at[pl.ds(base, _EPW)], v_idx)

  @pl.when(lax.axis_index("s") == 0)
  def _stage():
    pltpu.sync_copy(x_hbm, xs)

  plsc.subcore_barrier()

  rc = [relv[j] for j in range(4)]
  rs = [relv[4 + j] for j in range(4)]
  lane = lax.iota(jnp.int32, _L)
  perms = [lane ^ k for k in (8, 4, 2, 1)]

  def chunk(ci, _):
    off = ci * _CHUNK
    g1 = pltpu.async_copy(xs.at[u_idx.at[pl.ds(off, _CHUNK)]], ru, sem)
    g2 = pltpu.async_copy(xs.at[v_idx.at[pl.ds(off, _CHUNK)]], rv, sem)
    g1.wait()
    g2.wait()

    def group(g, _):
      def edge16(k, svec):
        e = g * _L + k
        acc = None
        for j in range(4):
          reu = ru[e, pl.ds(_L * j, _L)]
          imu = ru[e, pl.ds(_DIM_R + _L * j, _L)]
          rev = rv[e, pl.ds(_L * j, _L)]
          imv = rv[e, pl.ds(_DIM_R + _L * j, _L)]
          res = reu * rc[j] - imu * rs[j] - rev
          ims = imu * rc[j] + reu * rs[j] - imv
          t = _fast_sqrt(res * res + ims * ims)
          acc = t if acc is None else acc + t
        for p in perms:
          acc = acc + acc.at[p].get(mode="promise_in_bounds")
        return jnp.where(lane == k, acc, svec)

      svec = lax.fori_loop(0, _L, edge16, jnp.zeros((_L,), jnp.float32))
      ov[pl.ds(g * _L, _L)] = svec
      return 0

    lax.fori_loop(0, _CHUNK // _L, group, 0)
    pltpu.sync_copy(ov, out_hbm.at[pl.ds(base + off, _CHUNK)])
    return 0

  lax.fori_loop(0, _NCHUNK, chunk, 0)


@jax.jit
def kernel(x, edge_index, rel):
  u = edge_index[0].astype(jnp.int32)
  v = edge_index[1].astype(jnp.int32)
  r = rel.reshape(-1).astype(jnp.float32) / _PI
  relbuf = jnp.concatenate([jnp.cos(r), jnp.sin(r)]).reshape(8, _L)

  mesh = plsc.VectorSubcoreMesh(core_axis_name="c", subcore_axis_name="s")
  f = pl.kernel(
      _sc_body,
      out_type=jax.ShapeDtypeStruct((_N_EDGES,), jnp.float32),
      mesh=mesh,
      scratch_types=[
          pltpu.VMEM_SHARED((_N_NODES, _DIM), jnp.float32),
          pltpu.VMEM((_EPW,), jnp.int32),
          pltpu.VMEM((_EPW,), jnp.int32),
          pltpu.VMEM((_CHUNK, _DIM), jnp.float32),
          pltpu.VMEM((_CHUNK, _DIM), jnp.float32),
          pltpu.VMEM((8, _L), jnp.float32),
          pltpu.VMEM((_CHUNK,), jnp.float32),
          pltpu.SemaphoreType.DMA,
      ],
  )
  return f(x, u, v, relbuf)


# speedup vs baseline: 10.4397x; 1.3549x over previous
import jax
import jax.numpy as jnp
from jax import lax
from jax.experimental import pallas as pl
from jax.experimental.pallas import tpu as pltpu
from jax.experimental.pallas import tpu_sc as plsc

_N_NODES = 10000
_N_EDGES = 320000
_DIM = 128
_DIM_R = 64
_PI = 3.141592653589793

_NC = 2
_NS = 16
_NW = _NC * _NS
_L = 16

_EPW = _N_EDGES // _NW
_CHUNK = 80
_NCHUNK = _EPW // _CHUNK


def _fast_sqrt(s):
  i = lax.bitcast_convert_type(s, jnp.int32)
  i = jnp.int32(0x5F3759DF) - lax.shift_right_logical(i, 1)
  y = lax.bitcast_convert_type(i, jnp.float32)
  h = s * 0.5
  y = y * (1.5 - h * y * y)
  return s * y


def _sc_body(x_hbm, uv_hbm, rel_hbm, out_hbm,
             xs, ixa, ixb, au, av, bu, bv, relv, ova, ovb,
             sem_ga, sem_gb, sem_ix, sem_oa, sem_ob):
  wid = lax.axis_index("s") * _NC + lax.axis_index("c")
  base = wid * _EPW
  pltpu.sync_copy(rel_hbm, relv)

  @pl.when(lax.axis_index("s") == 0)
  def _stage():
    pltpu.sync_copy(x_hbm, xs)

  plsc.subcore_barrier()

  rc = [relv[j] for j in range(4)]
  rs = [relv[4 + j] for j in range(4)]
  lane = lax.iota(jnp.int32, _L)
  perms = [lane ^ k for k in (8, 4, 2, 1)]

  def boff(c):
    return (wid * _NCHUNK + c) * (2 * _CHUNK)

  def issue_gathers(ix, ru, rv, sem):
    pltpu.async_copy(xs.at[ix.at[pl.ds(0, _CHUNK)]], ru, sem)
    pltpu.async_copy(xs.at[ix.at[pl.ds(_CHUNK, _CHUNK)]], rv, sem)

  def drain_gathers(ix, ru, rv, sem):
    pltpu.make_async_copy(xs.at[ix.at[pl.ds(0, _CHUNK)]], ru, sem).wait()
    pltpu.make_async_copy(xs.at[ix.at[pl.ds(_CHUNK, _CHUNK)]], rv, sem).wait()

  def compute(ru, rv, ovx):
    def group(g, _):
      def edge16(k, svec):
        e = g * _L + k
        acc = None
        for j in range(4):
          reu = ru[e, pl.ds(_L * j, _L)]
          imu = ru[e, pl.ds(_DIM_R + _L * j, _L)]
          rev = rv[e, pl.ds(_L * j, _L)]
          imv = rv[e, pl.ds(_DIM_R + _L * j, _L)]
          res = reu * rc[j] - imu * rs[j] - rev
          ims = imu * rc[j] + reu * rs[j] - imv
          t = _fast_sqrt(res * res + ims * ims)
          acc = t if acc is None else acc + t
        for p in perms:
          acc = acc + acc.at[p].get(mode="promise_in_bounds")
        return jnp.where(lane == k, acc, svec)

      svec = lax.fori_loop(0, _L, edge16, jnp.zeros((_L,), jnp.float32))
      ovx[pl.ds(g * _L, _L)] = svec
      return 0

    lax.fori_loop(0, _CHUNK // _L, group, 0)

  pltpu.sync_copy(uv_hbm.at[pl.ds(boff(0), 2 * _CHUNK)], ixa)
  issue_gathers(ixa, au, av, sem_ga)
  pltpu.sync_copy(uv_hbm.at[pl.ds(boff(1), 2 * _CHUNK)], ixb)

  def dstep(i, _):
    c0 = 2 * i
    issue_gathers(ixb, bu, bv, sem_gb)
    hix = pltpu.async_copy(uv_hbm.at[pl.ds(boff(c0 + 2), 2 * _CHUNK)], ixa,
                           sem_ix)
    drain_gathers(ixa, au, av, sem_ga)
    compute(au, av, ova)
    ho_a = pltpu.async_copy(ova, out_hbm.at[pl.ds(base + c0 * _CHUNK, _CHUNK)],
                            sem_oa)
    hix.wait()
    issue_gathers(ixa, au, av, sem_ga)
    drain_gathers(ixb, bu, bv, sem_gb)
    compute(bu, bv, ovb)
    ho_b = pltpu.async_copy(ovb, out_hbm.at[pl.ds(base + (c0 + 1) * _CHUNK,
                                                  _CHUNK)], sem_ob)
    hib = pltpu.async_copy(uv_hbm.at[pl.ds(boff(c0 + 3), 2 * _CHUNK)], ixb,
                           sem_ix)
    ho_a.wait()
    ho_b.wait()
    hib.wait()
    return 0

  lax.fori_loop(0, (_NCHUNK - 3) // 2, dstep, 0)

  c0 = _NCHUNK - 3
  issue_gathers(ixb, bu, bv, sem_gb)
  drain_gathers(ixa, au, av, sem_ga)
  compute(au, av, ova)
  pltpu.sync_copy(ova, out_hbm.at[pl.ds(base + c0 * _CHUNK, _CHUNK)])
  pltpu.sync_copy(uv_hbm.at[pl.ds(boff(c0 + 2), 2 * _CHUNK)], ixa)
  issue_gathers(ixa, au, av, sem_ga)
  drain_gathers(ixb, bu, bv, sem_gb)
  compute(bu, bv, ovb)
  pltpu.sync_copy(ovb, out_hbm.at[pl.ds(base + (c0 + 1) * _CHUNK, _CHUNK)])
  drain_gathers(ixa, au, av, sem_ga)
  compute(au, av, ova)
  pltpu.sync_copy(ova, out_hbm.at[pl.ds(base + (c0 + 2) * _CHUNK, _CHUNK)])


@jax.jit
def kernel(x, edge_index, rel):
  u = edge_index[0].astype(jnp.int32).reshape(_NW, _NCHUNK, _CHUNK)
  v = edge_index[1].astype(jnp.int32).reshape(_NW, _NCHUNK, _CHUNK)
  uv = jnp.stack([u, v], axis=2).reshape(-1)
  r = rel.reshape(-1).astype(jnp.float32) / _PI
  relbuf = jnp.concatenate([jnp.cos(r), jnp.sin(r)]).reshape(8, _L)

  mesh = plsc.VectorSubcoreMesh(core_axis_name="c", subcore_axis_name="s")
  f = pl.kernel(
      _sc_body,
      out_type=jax.ShapeDtypeStruct((_N_EDGES,), jnp.float32),
      mesh=mesh,
      scratch_types=[
          pltpu.VMEM_SHARED((_N_NODES, _DIM), jnp.float32),
          pltpu.VMEM((2 * _CHUNK,), jnp.int32),
          pltpu.VMEM((2 * _CHUNK,), jnp.int32),
          pltpu.VMEM((_CHUNK, _DIM), jnp.float32),
          pltpu.VMEM((_CHUNK, _DIM), jnp.float32),
          pltpu.VMEM((_CHUNK, _DIM), jnp.float32),
          pltpu.VMEM((_CHUNK, _DIM), jnp.float32),
          pltpu.VMEM((8, _L), jnp.float32),
          pltpu.VMEM((_CHUNK,), jnp.float32),
          pltpu.VMEM((_CHUNK,), jnp.float32),
          pltpu.SemaphoreType.DMA,
          pltpu.SemaphoreType.DMA,
          pltpu.SemaphoreType.DMA,
          pltpu.SemaphoreType.DMA,
          pltpu.SemaphoreType.DMA,
      ],
  )
  return f(x, uv, relbuf)


# speedup vs baseline: 10.8840x; 1.0426x over previous
import jax
import jax.numpy as jnp
from jax import lax
from jax.experimental import pallas as pl
from jax.experimental.pallas import tpu as pltpu
from jax.experimental.pallas import tpu_sc as plsc

_N_NODES = 10000
_N_EDGES = 320000
_DIM = 128
_DIM_R = 64
_PI = 3.141592653589793

_NC = 2
_NS = 16
_NW = _NC * _NS
_L = 16

_EPW = _N_EDGES // _NW
_CHUNK = 80
_NCHUNK = _EPW // _CHUNK


def _fast_sqrt(s):
  i = lax.bitcast_convert_type(s, jnp.int32)
  i = jnp.int32(0x5F3759DF) - lax.shift_right_logical(i, 1)
  y = lax.bitcast_convert_type(i, jnp.float32)
  h = s * 0.5
  y = y * (1.5 - h * y * y)
  return s * y


def _sc_body(x_hbm, uv_hbm, rel_hbm, out_hbm,
             xs, ixa, ixb, au, av, bu, bv, relv, ova, ovb,
             sem_ga, sem_gb, sem_ix, sem_oa, sem_ob):
  wid = lax.axis_index("s") * _NC + lax.axis_index("c")
  base = wid * _EPW
  pltpu.sync_copy(rel_hbm, relv)

  @pl.when(lax.axis_index("s") == 0)
  def _stage():
    pltpu.sync_copy(x_hbm, xs)

  plsc.subcore_barrier()

  rc = [relv[j] for j in range(4)]
  rs = [relv[4 + j] for j in range(4)]
  lane = lax.iota(jnp.int32, _L)
  perms = [lane ^ k for k in (8, 4, 2, 1)]

  def boff(c):
    return (wid * _NCHUNK + c) * (2 * _CHUNK)

  def issue_gathers(ix, ru, rv, sem):
    pltpu.async_copy(xs.at[ix.at[pl.ds(0, _CHUNK)]], ru, sem)
    pltpu.async_copy(xs.at[ix.at[pl.ds(_CHUNK, _CHUNK)]], rv, sem)

  def drain_gathers(ix, ru, rv, sem):
    pltpu.make_async_copy(xs.at[ix.at[pl.ds(0, _CHUNK)]], ru, sem).wait()
    pltpu.make_async_copy(xs.at[ix.at[pl.ds(_CHUNK, _CHUNK)]], rv, sem).wait()

  def compute(ru, rv, ovx):
    def group(g, _):
      ebase = pl.multiple_of(g * _L, _L)
      svec = None
      for k in range(_L):
        e = ebase + k
        acc = None
        for j in range(4):
          reu = ru[e, pl.ds(_L * j, _L)]
          imu = ru[e, pl.ds(_DIM_R + _L * j, _L)]
          rev = rv[e, pl.ds(_L * j, _L)]
          imv = rv[e, pl.ds(_DIM_R + _L * j, _L)]
          res = reu * rc[j] - imu * rs[j] - rev
          ims = imu * rc[j] + reu * rs[j] - imv
          t = _fast_sqrt(res * res + ims * ims)
          acc = t if acc is None else acc + t
        for p in perms:
          acc = acc + acc.at[p].get(mode="promise_in_bounds")
        svec = acc if svec is None else jnp.where(lane == k, acc, svec)

      ovx[pl.ds(g * _L, _L)] = svec
      return 0

    lax.fori_loop(0, _CHUNK // _L, group, 0)

  pltpu.sync_copy(uv_hbm.at[pl.ds(boff(0), 2 * _CHUNK)], ixa)
  issue_gathers(ixa, au, av, sem_ga)
  pltpu.sync_copy(uv_hbm.at[pl.ds(boff(1), 2 * _CHUNK)], ixb)

  def dstep(i, _):
    c0 = 2 * i
    issue_gathers(ixb, bu, bv, sem_gb)
    hix = pltpu.async_copy(uv_hbm.at[pl.ds(boff(c0 + 2), 2 * _CHUNK)], ixa,
                           sem_ix)
    drain_gathers(ixa, au, av, sem_ga)
    compute(au, av, ova)
    ho_a = pltpu.async_copy(ova, out_hbm.at[pl.ds(base + c0 * _CHUNK, _CHUNK)],
                            sem_oa)
    hix.wait()
    issue_gathers(ixa, au, av, sem_ga)
    drain_gathers(ixb, bu, bv, sem_gb)
    compute(bu, bv, ovb)
    ho_b = pltpu.async_copy(ovb, out_hbm.at[pl.ds(base + (c0 + 1) * _CHUNK,
                                                  _CHUNK)], sem_ob)
    hib = pltpu.async_copy(uv_hbm.at[pl.ds(boff(c0 + 3), 2 * _CHUNK)], ixb,
                           sem_ix)
    ho_a.wait()
    ho_b.wait()
    hib.wait()
    return 0

  lax.fori_loop(0, (_NCHUNK - 3) // 2, dstep, 0)

  c0 = _NCHUNK - 3
  issue_gathers(ixb, bu, bv, sem_gb)
  drain_gathers(ixa, au, av, sem_ga)
  compute(au, av, ova)
  pltpu.sync_copy(ova, out_hbm.at[pl.ds(base + c0 * _CHUNK, _CHUNK)])
  pltpu.sync_copy(uv_hbm.at[pl.ds(boff(c0 + 2), 2 * _CHUNK)], ixa)
  issue_gathers(ixa, au, av, sem_ga)
  drain_gathers(ixb, bu, bv, sem_gb)
  compute(bu, bv, ovb)
  pltpu.sync_copy(ovb, out_hbm.at[pl.ds(base + (c0 + 1) * _CHUNK, _CHUNK)])
  drain_gathers(ixa, au, av, sem_ga)
  compute(au, av, ova)
  pltpu.sync_copy(ova, out_hbm.at[pl.ds(base + (c0 + 2) * _CHUNK, _CHUNK)])


@jax.jit
def kernel(x, edge_index, rel):
  u = edge_index[0].astype(jnp.int32).reshape(_NW, _NCHUNK, _CHUNK)
  v = edge_index[1].astype(jnp.int32).reshape(_NW, _NCHUNK, _CHUNK)
  uv = jnp.stack([u, v], axis=2).reshape(-1)
  r = rel.reshape(-1).astype(jnp.float32) / _PI
  relbuf = jnp.concatenate([jnp.cos(r), jnp.sin(r)]).reshape(8, _L)

  mesh = plsc.VectorSubcoreMesh(core_axis_name="c", subcore_axis_name="s")
  f = pl.kernel(
      _sc_body,
      out_type=jax.ShapeDtypeStruct((_N_EDGES,), jnp.float32),
      mesh=mesh,
      scratch_types=[
          pltpu.VMEM_SHARED((_N_NODES, _DIM), jnp.float32),
          pltpu.VMEM((2 * _CHUNK,), jnp.int32),
          pltpu.VMEM((2 * _CHUNK,), jnp.int32),
          pltpu.VMEM((_CHUNK, _DIM), jnp.float32),
          pltpu.VMEM((_CHUNK, _DIM), jnp.float32),
          pltpu.VMEM((_CHUNK, _DIM), jnp.float32),
          pltpu.VMEM((_CHUNK, _DIM), jnp.float32),
          pltpu.VMEM((8, _L), jnp.float32),
          pltpu.VMEM((_CHUNK,), jnp.float32),
          pltpu.VMEM((_CHUNK,), jnp.float32),
          pltpu.SemaphoreType.DMA,
          pltpu.SemaphoreType.DMA,
          pltpu.SemaphoreType.DMA,
          pltpu.SemaphoreType.DMA,
          pltpu.SemaphoreType.DMA,
      ],
  )
  return f(x, uv, relbuf)


# speedup vs baseline: 13.0289x; 1.1971x over previous
import jax
import jax.numpy as jnp
from jax import lax
from jax.experimental import pallas as pl
from jax.experimental.pallas import tpu as pltpu
from jax.experimental.pallas import tpu_sc as plsc

_N_NODES = 10000
_N_EDGES = 320000
_DIM = 128
_DIM_R = 64
_PI = 3.141592653589793

_NC = 2
_NS = 16
_NW = _NC * _NS
_L = 16

_EPW = _N_EDGES // _NW
_CHUNK = 80
_NCHUNK = _EPW // _CHUNK


def _fast_sqrt_bf16(s):
  w = plsc.bitcast(s, jnp.int32)
  w = lax.shift_right_logical(w, 1) & jnp.int32(0x7FFF7FFF)
  w = jnp.int32(0x5F375F37) - w
  y = plsc.bitcast(w, jnp.bfloat16)
  h = s * jnp.bfloat16(0.5)
  y = y * (jnp.bfloat16(1.5) - h * y * y)
  return s * y


def _sc_body(x_hbm, uv_hbm, rel_hbm, out_hbm,
             xs, ixa, ixb, au, av, bu, bv, relv, ova, ovb,
             sem_ga, sem_gb, sem_ix, sem_oa, sem_ob):
  wid = lax.axis_index("s") * _NC + lax.axis_index("c")
  base = wid * _EPW
  pltpu.sync_copy(rel_hbm, relv)

  @pl.when(lax.axis_index("s") == 0)
  def _stage():
    pltpu.sync_copy(x_hbm, xs)

  plsc.subcore_barrier()

  rc = [plsc.bitcast(relv[j], jnp.bfloat16) for j in range(2)]
  rs = [plsc.bitcast(relv[2 + j], jnp.bfloat16) for j in range(2)]
  lane = lax.iota(jnp.int32, _L)
  perms = [lane ^ k for k in (8, 4, 2, 1)]

  def boff(c):
    return (wid * _NCHUNK + c) * (2 * _CHUNK)

  def issue_gathers(ix, ru, rv, sem):
    pltpu.async_copy(xs.at[ix.at[pl.ds(0, _CHUNK)]], ru, sem)
    pltpu.async_copy(xs.at[ix.at[pl.ds(_CHUNK, _CHUNK)]], rv, sem)

  def drain_gathers(ix, ru, rv, sem):
    pltpu.make_async_copy(xs.at[ix.at[pl.ds(0, _CHUNK)]], ru, sem).wait()
    pltpu.make_async_copy(xs.at[ix.at[pl.ds(_CHUNK, _CHUNK)]], rv, sem).wait()

  def compute(ru, rv, ovx):
    def group(g, _):
      ebase = pl.multiple_of(g * _L, _L)
      svec = None
      for k in range(_L):
        e = ebase + k
        acc = None
        for j in range(2):
          pk = plsc.PackFormat.INTERLEAVED
          reu = plsc.pack(ru[e, pl.ds(2 * _L * j, _L)],
                          ru[e, pl.ds(2 * _L * j + _L, _L)], format=pk)
          imu = plsc.pack(ru[e, pl.ds(_DIM_R + 2 * _L * j, _L)],
                          ru[e, pl.ds(_DIM_R + 2 * _L * j + _L, _L)], format=pk)
          rev = plsc.pack(rv[e, pl.ds(2 * _L * j, _L)],
                          rv[e, pl.ds(2 * _L * j + _L, _L)], format=pk)
          imv = plsc.pack(rv[e, pl.ds(_DIM_R + 2 * _L * j, _L)],
                          rv[e, pl.ds(_DIM_R + 2 * _L * j + _L, _L)], format=pk)
          res = reu * rc[j] - imu * rs[j] - rev
          ims = imu * rc[j] + reu * rs[j] - imv
          t = _fast_sqrt_bf16(res * res + ims * ims)
          lo, hi = plsc.unpack(t, format=pk)
          acc = lo + hi if acc is None else acc + lo + hi
        for p in perms:
          acc = acc + acc.at[p].get(mode="promise_in_bounds")
        svec = acc if svec is None else jnp.where(lane == k, acc, svec)

      ovx[pl.ds(g * _L, _L)] = svec
      return 0

    lax.fori_loop(0, _CHUNK // _L, group, 0)

  pltpu.sync_copy(uv_hbm.at[pl.ds(boff(0), 2 * _CHUNK)], ixa)
  issue_gathers(ixa, au, av, sem_ga)
  pltpu.sync_copy(uv_hbm.at[pl.ds(boff(1), 2 * _CHUNK)], ixb)

  def dstep(i, _):
    c0 = 2 * i
    issue_gathers(ixb, bu, bv, sem_gb)
    hix = pltpu.async_copy(uv_hbm.at[pl.ds(boff(c0 + 2), 2 * _CHUNK)], ixa,
                           sem_ix)
    drain_gathers(ixa, au, av, sem_ga)
    compute(au, av, ova)
    ho_a = pltpu.async_copy(ova, out_hbm.at[pl.ds(base + c0 * _CHUNK, _CHUNK)],
                            sem_oa)
    hix.wait()
    issue_gathers(ixa, au, av, sem_ga)
    drain_gathers(ixb, bu, bv, sem_gb)
    compute(bu, bv, ovb)
    ho_b = pltpu.async_copy(ovb, out_hbm.at[pl.ds(base + (c0 + 1) * _CHUNK,
                                                  _CHUNK)], sem_ob)
    hib = pltpu.async_copy(uv_hbm.at[pl.ds(boff(c0 + 3), 2 * _CHUNK)], ixb,
                           sem_ix)
    ho_a.wait()
    ho_b.wait()
    hib.wait()
    return 0

  lax.fori_loop(0, (_NCHUNK - 3) // 2, dstep, 0)

  c0 = _NCHUNK - 3
  issue_gathers(ixb, bu, bv, sem_gb)
  drain_gathers(ixa, au, av, sem_ga)
  compute(au, av, ova)
  pltpu.sync_copy(ova, out_hbm.at[pl.ds(base + c0 * _CHUNK, _CHUNK)])
  pltpu.sync_copy(uv_hbm.at[pl.ds(boff(c0 + 2), 2 * _CHUNK)], ixa)
  issue_gathers(ixa, au, av, sem_ga)
  drain_gathers(ixb, bu, bv, sem_gb)
  compute(bu, bv, ovb)
  pltpu.sync_copy(ovb, out_hbm.at[pl.ds(base + (c0 + 1) * _CHUNK, _CHUNK)])
  drain_gathers(ixa, au, av, sem_ga)
  compute(au, av, ova)
  pltpu.sync_copy(ova, out_hbm.at[pl.ds(base + (c0 + 2) * _CHUNK, _CHUNK)])


@jax.jit
def kernel(x, edge_index, rel):
  u = edge_index[0].astype(jnp.int32).reshape(_NW, _NCHUNK, _CHUNK)
  v = edge_index[1].astype(jnp.int32).reshape(_NW, _NCHUNK, _CHUNK)
  uv = jnp.stack([u, v], axis=2).reshape(-1)
  r = rel.reshape(-1).astype(jnp.float32) / _PI
  c, s = jnp.cos(r), jnp.sin(r)
  def _inter(a):
    a = a.reshape(2, 2, _L)
    return jnp.stack([a[:, 0], a[:, 1]], axis=-1)
  cs = jnp.concatenate([_inter(c), _inter(s)]).astype(jnp.bfloat16)
  relbuf = lax.bitcast_convert_type(cs, jnp.int32)

  mesh = plsc.VectorSubcoreMesh(core_axis_name="c", subcore_axis_name="s")
  f = pl.kernel(
      _sc_body,
      out_type=jax.ShapeDtypeStruct((_N_EDGES,), jnp.float32),
      mesh=mesh,
      compiler_params=pltpu.CompilerParams(needs_layout_passes=False),
      scratch_types=[
          pltpu.VMEM_SHARED((_N_NODES, _DIM), jnp.float32),
          pltpu.VMEM((2 * _CHUNK,), jnp.int32),
          pltpu.VMEM((2 * _CHUNK,), jnp.int32),
          pltpu.VMEM((_CHUNK, _DIM), jnp.float32),
          pltpu.VMEM((_CHUNK, _DIM), jnp.float32),
          pltpu.VMEM((_CHUNK, _DIM), jnp.float32),
          pltpu.VMEM((_CHUNK, _DIM), jnp.float32),
          pltpu.VMEM((4, _L), jnp.int32),
          pltpu.VMEM((_CHUNK,), jnp.float32),
          pltpu.VMEM((_CHUNK,), jnp.float32),
          pltpu.SemaphoreType.DMA,
          pltpu.SemaphoreType.DMA,
          pltpu.SemaphoreType.DMA,
          pltpu.SemaphoreType.DMA,
          pltpu.SemaphoreType.DMA,
      ],
  )
  return f(x, uv, relbuf)
